# baseline (speedup 1.0000x reference)
import functools

import jax
import jax.numpy as jnp
from jax import lax
from jax.experimental import pallas as pl
from jax.experimental.pallas import tpu as pltpu
from jax.experimental.pallas import tpu_sc as plsc

L = 10000
E = 320000
D = 128
H = 4
DH = D // H
SCALE = DH ** -0.5

LP = 10240
NTILES = 32
CHUNK = 128
ROWS_PER_TILE = LP // 16
HP = 16
SMW = LP * H
CHUNK_SM = 768

E1 = 159744
E2 = E - E1

_PREC = lax.Precision.HIGHEST


def _proj_body(x_ref, w_ref, b_ref, o_ref):
    o_ref[...] = (
        jnp.dot(x_ref[...], w_ref[...], precision=_PREC) + b_ref[...]
    )


def _project(x, w, b, block):
    n = x.shape[0]
    grid = n // block
    return pl.pallas_call(
        _proj_body,
        grid=(grid,),
        in_specs=[
            pl.BlockSpec((block, D), lambda i: (i, 0)),
            pl.BlockSpec((D, D), lambda i: (0, 0)),
            pl.BlockSpec((1, D), lambda i: (0, 0)),
        ],
        out_specs=pl.BlockSpec((block, D), lambda i: (i, 0)),
        out_shape=jax.ShapeDtypeStruct((n, D), jnp.float32),
    )(x, w, b)


def _gather_rows(q, idx, ept):
    nfull = ept // CHUNK
    tail = ept - nfull * CHUNK

    def body(q_hbm, idx_hbm, qe_hbm, idx_a, idx_b, rows_a, rows_b,
             s0, s1, s2, s3, s4, s5):
        wid = lax.axis_index("s") * 2 + lax.axis_index("c")
        base = wid * ept

        def pair(j, carry):
            off_a = base + (2 * j) * CHUNK
            off_b = off_a + CHUNK
            ca = pltpu.async_copy(idx_hbm.at[pl.ds(off_a, CHUNK)], idx_a, s0)
            cb = pltpu.async_copy(idx_hbm.at[pl.ds(off_b, CHUNK)], idx_b, s1)
            ca.wait()
            ga = pltpu.async_copy(q_hbm.at[idx_a], rows_a, s2)
            cb.wait()
            gb = pltpu.async_copy(q_hbm.at[idx_b], rows_b, s3)
            ga.wait()
            wa = pltpu.async_copy(rows_a, qe_hbm.at[pl.ds(off_a, CHUNK)], s4)
            gb.wait()
            wb = pltpu.async_copy(rows_b, qe_hbm.at[pl.ds(off_b, CHUNK)], s5)
            wa.wait()
            wb.wait()
            return carry

        lax.fori_loop(0, nfull // 2, pair, 0)

        if nfull % 2:
            off = base + (nfull - 1) * CHUNK
            pltpu.sync_copy(idx_hbm.at[pl.ds(off, CHUNK)], idx_a)
            pltpu.async_copy(q_hbm.at[idx_a], rows_a, s0).wait()
            pltpu.sync_copy(rows_a, qe_hbm.at[pl.ds(off, CHUNK)])
        if tail:
            off = base + nfull * CHUNK
            pltpu.sync_copy(idx_hbm.at[pl.ds(off, tail)],
                            idx_b.at[pl.ds(0, tail)])
            pltpu.async_copy(q_hbm.at[idx_b.at[pl.ds(0, tail)]],
                             rows_b.at[pl.ds(0, tail)], s1).wait()
            pltpu.sync_copy(rows_b.at[pl.ds(0, tail)],
                            qe_hbm.at[pl.ds(off, tail)])

    mesh = plsc.VectorSubcoreMesh(core_axis_name="c", subcore_axis_name="s")
    kern = functools.partial(
        pl.kernel,
        mesh=mesh,
        out_type=jax.ShapeDtypeStruct((ept * NTILES, D), jnp.float32),
        scratch_types=[
            pltpu.VMEM((CHUNK,), jnp.int32),
            pltpu.VMEM((CHUNK,), jnp.int32),
            pltpu.VMEM((CHUNK, D), jnp.float32),
            pltpu.VMEM((CHUNK, D), jnp.float32),
            pltpu.SemaphoreType.DMA,
            pltpu.SemaphoreType.DMA,
            pltpu.SemaphoreType.DMA,
            pltpu.SemaphoreType.DMA,
            pltpu.SemaphoreType.DMA,
            pltpu.SemaphoreType.DMA,
        ],
    )(body)
    return kern(q, idx)


def _edge_body(k_ref, v_ref, qe_ref, wk_ref, bk_ref, wv_ref, bv_ref,
               vvu_ref, exw_ref):
    kk = jnp.dot(k_ref[...], wk_ref[...]) + bk_ref[...]
    vv = jnp.dot(v_ref[...], wv_ref[...]) + bv_ref[...]
    prod = qe_ref[...] * kk
    d_ids = lax.broadcasted_iota(jnp.int32, (D, HP), 0) // DH
    h_ids = lax.broadcasted_iota(jnp.int32, (D, HP), 1)
    g = jnp.where(d_ids == h_ids, 1.0, 0.0).astype(jnp.float32)
    logits = jnp.dot(prod, g, precision=_PREC) * SCALE
    col = lax.broadcasted_iota(jnp.int32, logits.shape, 1)
    ex = jnp.where(col < H, jnp.exp(logits), 0.0)
    exw_ref[...] = ex
    b_ids = lax.broadcasted_iota(jnp.int32, (HP, D), 0)
    d2_ids = lax.broadcasted_iota(jnp.int32, (HP, D), 1) // DH
    expand = jnp.where(b_ids == d2_ids, 1.0, 0.0).astype(jnp.float32)
    mult = jnp.dot(ex, expand, precision=_PREC)
    vvu_ref[...] = vv * mult


def _edge_stage(k, v, qe, wk, bk, wv, bv, block=512):
    n = k.shape[0]
    grid = n // block
    return pl.pallas_call(
        _edge_body,
        grid=(grid,),
        in_specs=[
            pl.BlockSpec((block, D), lambda i: (i, 0)),
            pl.BlockSpec((block, D), lambda i: (i, 0)),
            pl.BlockSpec((block, D), lambda i: (i, 0)),
            pl.BlockSpec((D, D), lambda i: (0, 0)),
            pl.BlockSpec((1, D), lambda i: (0, 0)),
            pl.BlockSpec((D, D), lambda i: (0, 0)),
            pl.BlockSpec((1, D), lambda i: (0, 0)),
        ],
        out_specs=[
            pl.BlockSpec((block, D), lambda i: (i, 0)),
            pl.BlockSpec((block, HP), lambda i: (i, 0)),
        ],
        out_shape=[
            jax.ShapeDtypeStruct((n, D), jnp.float32),
            jax.ShapeDtypeStruct((n, HP), jnp.float32),
        ],
    )(k, v, qe, wk, bk, wv, bv)


def _scatter_agg(vvu, idx, ept):
    nfull = ept // CHUNK
    tail = ept - nfull * CHUNK

    def body(vvu_hbm, idx_hbm, zrow_hbm, agg_hbm,
             idx_a, idx_b, rows_a, rows_b, idx_t, rows_t, agg_sh,
             s0, s1, s2, s3, s4, s5):
        cid = lax.axis_index("c")
        sid = lax.axis_index("s")
        wid = sid * 2 + cid
        base = wid * ept
        row0 = sid * ROWS_PER_TILE

        for z in range(ROWS_PER_TILE // CHUNK):
            pltpu.sync_copy(zrow_hbm, agg_sh.at[pl.ds(row0 + z * CHUNK, CHUNK)])
        plsc.subcore_barrier()

        def pair(j, carry):
            off_a = base + (2 * j) * CHUNK
            off_b = off_a + CHUNK
            ia = pltpu.async_copy(idx_hbm.at[pl.ds(off_a, CHUNK)], idx_a, s0)
            ib = pltpu.async_copy(idx_hbm.at[pl.ds(off_b, CHUNK)], idx_b, s1)
            ra = pltpu.async_copy(vvu_hbm.at[pl.ds(off_a, CHUNK)], rows_a, s2)
            rb = pltpu.async_copy(vvu_hbm.at[pl.ds(off_b, CHUNK)], rows_b, s3)
            ia.wait()
            ra.wait()
            sa = pltpu.async_copy(rows_a, agg_sh.at[idx_a], s4, add=True)
            ib.wait()
            rb.wait()
            sb = pltpu.async_copy(rows_b, agg_sh.at[idx_b], s5, add=True)
            sa.wait()
            sb.wait()
            return carry

        lax.fori_loop(0, nfull // 2, pair, 0)

        if nfull % 2:
            off = base + (nfull - 1) * CHUNK
            pltpu.sync_copy(idx_hbm.at[pl.ds(off, CHUNK)], idx_a)
            pltpu.sync_copy(vvu_hbm.at[pl.ds(off, CHUNK)], rows_a)
            pltpu.sync_copy(rows_a, agg_sh.at[idx_a], add=True)
        if tail:
            off = base + nfull * CHUNK
            pltpu.sync_copy(idx_hbm.at[pl.ds(off, tail)], idx_t)
            pltpu.sync_copy(vvu_hbm.at[pl.ds(off, tail)], rows_t)
            pltpu.sync_copy(rows_t, agg_sh.at[idx_t], add=True)

        plsc.subcore_barrier()
        pltpu.sync_copy(agg_sh.at[pl.ds(row0, ROWS_PER_TILE)],
                        agg_hbm.at[cid, pl.ds(row0, ROWS_PER_TILE)])

    mesh = plsc.VectorSubcoreMesh(core_axis_name="c", subcore_axis_name="s")
    zrow = jnp.zeros((CHUNK, D), jnp.float32)
    kern = functools.partial(
        pl.kernel,
        mesh=mesh,
        out_type=jax.ShapeDtypeStruct((2, LP, D), jnp.float32),
        scratch_types=[
            pltpu.VMEM((CHUNK,), jnp.int32),
            pltpu.VMEM((CHUNK,), jnp.int32),
            pltpu.VMEM((CHUNK, D), jnp.float32),
            pltpu.VMEM((CHUNK, D), jnp.float32),
            pltpu.VMEM((16,), jnp.int32),
            pltpu.VMEM((16, D), jnp.float32),
            pltpu.VMEM_SHARED((LP, D), jnp.float32),
            pltpu.SemaphoreType.DMA,
            pltpu.SemaphoreType.DMA,
            pltpu.SemaphoreType.DMA,
            pltpu.SemaphoreType.DMA,
            pltpu.SemaphoreType.DMA,
            pltpu.SemaphoreType.DMA,
        ],
    )(body)
    return kern(vvu, idx, zrow)


def _scatter_sm(exw, idx, ept):
    nfull = ept // CHUNK_SM
    tail = ept - nfull * CHUNK_SM

    def body(exw_hbm, idx_hbm, sm_hbm, idx_v, ex_v, sm_t, sem):
        cid = lax.axis_index("c")
        sid = lax.axis_index("s")
        wid = sid * 2 + cid
        base = wid * ept

        def zero(i, carry):
            sm_t[pl.ds(i * 16, 16)] = jnp.zeros((16,), jnp.float32)
            return carry

        lax.fori_loop(0, SMW // 16, zero, 0)

        def scatter_sm(nedges):
            for g in range(nedges // 16):
                rows = lax.iota(jnp.int32, 16) + g * 16
                nidx = idx_v[pl.ds(g * 16, 16)]
                for h in range(H):
                    vals = plsc.load_gather(ex_v, [rows * HP + h])
                    plsc.addupdate_scatter(sm_t, [nidx * H + h], vals)

        def loop(c, carry):
            off = base + c * CHUNK_SM
            pltpu.sync_copy(idx_hbm.at[pl.ds(off, CHUNK_SM)], idx_v)
            pltpu.sync_copy(exw_hbm.at[pl.ds(off * HP, CHUNK_SM * HP)], ex_v)
            scatter_sm(CHUNK_SM)
            return carry

        lax.fori_loop(0, nfull, loop, 0)

        if tail:
            off = base + nfull * CHUNK_SM
            pltpu.sync_copy(idx_hbm.at[pl.ds(off, tail)],
                            idx_v.at[pl.ds(0, tail)])
            pltpu.sync_copy(exw_hbm.at[pl.ds(off * HP, tail * HP)],
                            ex_v.at[pl.ds(0, tail * HP)])
            scatter_sm(tail)

        pltpu.sync_copy(sm_t, sm_hbm.at[wid])

    mesh = plsc.VectorSubcoreMesh(core_axis_name="c", subcore_axis_name="s")
    kern = functools.partial(
        pl.kernel,
        mesh=mesh,
        compiler_params=pltpu.CompilerParams(needs_layout_passes=False),
        out_type=jax.ShapeDtypeStruct((NTILES, SMW), jnp.float32),
        scratch_types=[
            pltpu.VMEM((CHUNK_SM,), jnp.int32),
            pltpu.VMEM((CHUNK_SM * HP,), jnp.float32),
            pltpu.VMEM((SMW,), jnp.float32),
            pltpu.SemaphoreType.DMA,
        ],
    )(body)
    return kern(exw.reshape(-1), idx)


def _rsm_body(a_ref, b_ref, o_ref):
    s = jnp.sum(a_ref[...], axis=0) + jnp.sum(b_ref[...], axis=0)
    o_ref[...] = (1.0 / (s + 1e-7)).reshape(1, -1)


def _reduce_sm(sm_a, sm_b, block=8192):
    grid = SMW // block
    return pl.pallas_call(
        _rsm_body,
        grid=(grid,),
        in_specs=[
            pl.BlockSpec((NTILES, block), lambda i: (0, i)),
            pl.BlockSpec((NTILES, block), lambda i: (0, i)),
        ],
        out_specs=pl.BlockSpec((1, block), lambda i: (0, i)),
        out_shape=jax.ShapeDtypeStruct((1, SMW), jnp.float32),
    )(sm_a, sm_b)


def _final_body(agg1_ref, agg2_ref, r_ref, wo_ref, bo_ref,
                g_ref, b_ref, o_ref):
    agg = (agg1_ref[0] + agg1_ref[1]) + (agg2_ref[0] + agg2_ref[1])
    b_ids = lax.broadcasted_iota(jnp.int32, (H, D), 0)
    d_ids = lax.broadcasted_iota(jnp.int32, (H, D), 1) // DH
    expand = jnp.where(b_ids == d_ids, 1.0, 0.0).astype(jnp.float32)
    mult = jnp.dot(r_ref[...], expand, precision=_PREC)
    out = jnp.dot(agg * mult, wo_ref[...], precision=_PREC) + bo_ref[...]
    mean = jnp.mean(out, axis=-1, keepdims=True)
    var = jnp.mean((out - mean) ** 2, axis=-1, keepdims=True)
    o_ref[...] = (out - mean) / jnp.sqrt(var + 1e-5) * g_ref[...] + b_ref[...]


def _final_stage(agg_a, agg_b, recip, wo, bo, gamma, beta, block=1000):
    grid = L // block
    return pl.pallas_call(
        _final_body,
        grid=(grid,),
        in_specs=[
            pl.BlockSpec((2, block, D), lambda i: (0, i, 0)),
            pl.BlockSpec((2, block, D), lambda i: (0, i, 0)),
            pl.BlockSpec((block, H), lambda i: (i, 0)),
            pl.BlockSpec((D, D), lambda i: (0, 0)),
            pl.BlockSpec((1, D), lambda i: (0, 0)),
            pl.BlockSpec((1, D), lambda i: (0, 0)),
            pl.BlockSpec((1, D), lambda i: (0, 0)),
        ],
        out_specs=pl.BlockSpec((block, D), lambda i: (i, 0)),
        out_shape=jax.ShapeDtypeStruct((L, D), jnp.float32),
    )(agg_a, agg_b, recip, wo, bo, gamma, beta)


def kernel(f, k, v, agg_idx, Wq, bq, Wk, bk, Wv, bv, Wo, bo, gamma, beta):
    idx = agg_idx.astype(jnp.int32)
    bq2 = bq.reshape(1, D)
    bk2 = bk.reshape(1, D)
    bv2 = bv.reshape(1, D)
    bo2 = bo.reshape(1, D)
    g2 = gamma.reshape(1, D)
    be2 = beta.reshape(1, D)

    idx1, idx2 = idx[:E1], idx[E1:]
    ept1, ept2 = E1 // NTILES, E2 // NTILES

    q = _project(f, Wq, bq2, block=1000)
    qe1 = _gather_rows(q, idx1, ept1)
    vvu1, exw1 = _edge_stage(k[:E1], v[:E1], qe1, Wk, bk2, Wv, bv2)
    qe2 = _gather_rows(q, idx2, ept2)
    vvu2, exw2 = _edge_stage(k[E1:], v[E1:], qe2, Wk, bk2, Wv, bv2)
    agg_a = _scatter_agg(vvu1, idx1, ept1)
    sm_a = _scatter_sm(exw1, idx1, ept1)
    agg_b = _scatter_agg(vvu2, idx2, ept2)
    sm_b = _scatter_sm(exw2, idx2, ept2)

    recip = _reduce_sm(sm_a, sm_b)
    agg_a = agg_a[:, :L]
    agg_b = agg_b[:, :L]
    recip = recip.reshape(LP, H)[:L]
    return _final_stage(agg_a, agg_b, recip, Wo, bo2, g2, be2)


# speedup vs baseline: 1.1171x; 1.1171x over previous
import functools

import jax
import jax.numpy as jnp
from jax import lax
from jax.experimental import pallas as pl
from jax.experimental.pallas import tpu as pltpu
from jax.experimental.pallas import tpu_sc as plsc

L = 10000
E = 320000
D = 128
H = 4
DH = D // H
SCALE = DH ** -0.5

LP = 10240
NTILES = 32
EDGES_PER_TILE = E // NTILES
CHUNK = 128
NFULL = EDGES_PER_TILE // CHUNK
TAIL = EDGES_PER_TILE - NFULL * CHUNK
ROWS_PER_TILE = LP // 16
HP = 16

_PREC = lax.Precision.HIGHEST


def _proj_body(x_ref, w_ref, b_ref, o_ref):
    o_ref[...] = (
        jnp.dot(x_ref[...], w_ref[...], precision=_PREC) + b_ref[...]
    )


def _project(x, w, b, block):
    n = x.shape[0]
    grid = n // block
    return pl.pallas_call(
        _proj_body,
        grid=(grid,),
        in_specs=[
            pl.BlockSpec((block, D), lambda i: (i, 0)),
            pl.BlockSpec((D, D), lambda i: (0, 0)),
            pl.BlockSpec((1, D), lambda i: (0, 0)),
        ],
        out_specs=pl.BlockSpec((block, D), lambda i: (i, 0)),
        out_shape=jax.ShapeDtypeStruct((n, D), jnp.float32),
    )(x, w, b)


def _gather_body(q_hbm, idx_hbm, qe_hbm, idx_a, idx_b, rows_a, rows_b,
                 idx_t, rows_t, s0, s1, s2, s3, s4, s5):
    wid = lax.axis_index("s") * 2 + lax.axis_index("c")
    base = wid * EDGES_PER_TILE

    def pair(j, carry):
        off_a = base + (2 * j) * CHUNK
        off_b = off_a + CHUNK
        ca = pltpu.async_copy(idx_hbm.at[pl.ds(off_a, CHUNK)], idx_a, s0)
        cb = pltpu.async_copy(idx_hbm.at[pl.ds(off_b, CHUNK)], idx_b, s1)
        ca.wait()
        ga = pltpu.async_copy(q_hbm.at[idx_a], rows_a, s2)
        cb.wait()
        gb = pltpu.async_copy(q_hbm.at[idx_b], rows_b, s3)
        ga.wait()
        wa = pltpu.async_copy(rows_a, qe_hbm.at[pl.ds(off_a, CHUNK)], s4)
        gb.wait()
        wb = pltpu.async_copy(rows_b, qe_hbm.at[pl.ds(off_b, CHUNK)], s5)
        wa.wait()
        wb.wait()
        return carry

    lax.fori_loop(0, NFULL // 2, pair, 0)

    off = base + NFULL * CHUNK
    pltpu.sync_copy(idx_hbm.at[pl.ds(off, TAIL)], idx_t)
    pltpu.async_copy(q_hbm.at[idx_t], rows_t, s0).wait()
    pltpu.sync_copy(rows_t, qe_hbm.at[pl.ds(off, TAIL)])


def _gather_rows(q, idx):
    mesh = plsc.VectorSubcoreMesh(core_axis_name="c", subcore_axis_name="s")
    kern = functools.partial(
        pl.kernel,
        mesh=mesh,
        out_type=jax.ShapeDtypeStruct((E, D), jnp.float32),
        scratch_types=[
            pltpu.VMEM((CHUNK,), jnp.int32),
            pltpu.VMEM((CHUNK,), jnp.int32),
            pltpu.VMEM((CHUNK, D), jnp.float32),
            pltpu.VMEM((CHUNK, D), jnp.float32),
            pltpu.VMEM((TAIL,), jnp.int32),
            pltpu.VMEM((TAIL, D), jnp.float32),
            pltpu.SemaphoreType.DMA,
            pltpu.SemaphoreType.DMA,
            pltpu.SemaphoreType.DMA,
            pltpu.SemaphoreType.DMA,
            pltpu.SemaphoreType.DMA,
            pltpu.SemaphoreType.DMA,
        ],
    )(_gather_body)
    return kern(q, idx)


def _edge_body(k_ref, v_ref, qe_ref, wk_ref, bk_ref, wv_ref, bv_ref,
               vvu_ref, exw_ref):
    kk = jnp.dot(k_ref[...], wk_ref[...]) + bk_ref[...]
    vv = jnp.dot(v_ref[...], wv_ref[...]) + bv_ref[...]
    prod = qe_ref[...] * kk
    d_ids = lax.broadcasted_iota(jnp.int32, (D, HP), 0) // DH
    h_ids = lax.broadcasted_iota(jnp.int32, (D, HP), 1)
    g = jnp.where(d_ids == h_ids, 1.0, 0.0).astype(jnp.float32)
    logits = jnp.dot(prod, g, precision=_PREC) * SCALE
    col = lax.broadcasted_iota(jnp.int32, logits.shape, 1)
    ex = jnp.where(col < H, jnp.exp(logits), 0.0)
    exw_ref[...] = ex
    b_ids = lax.broadcasted_iota(jnp.int32, (HP, D), 0)
    d2_ids = lax.broadcasted_iota(jnp.int32, (HP, D), 1) // DH
    expand = jnp.where(b_ids == d2_ids, 1.0, 0.0).astype(jnp.float32)
    mult = jnp.dot(ex, expand, precision=_PREC)
    vvu_ref[...] = vv * mult


def _edge_stage(k, v, qe, wk, bk, wv, bv, block=512):
    grid = E // block
    return pl.pallas_call(
        _edge_body,
        grid=(grid,),
        in_specs=[
            pl.BlockSpec((block, D), lambda i: (i, 0)),
            pl.BlockSpec((block, D), lambda i: (i, 0)),
            pl.BlockSpec((block, D), lambda i: (i, 0)),
            pl.BlockSpec((D, D), lambda i: (0, 0)),
            pl.BlockSpec((1, D), lambda i: (0, 0)),
            pl.BlockSpec((D, D), lambda i: (0, 0)),
            pl.BlockSpec((1, D), lambda i: (0, 0)),
        ],
        out_specs=[
            pl.BlockSpec((block, D), lambda i: (i, 0)),
            pl.BlockSpec((block, HP), lambda i: (i, 0)),
        ],
        out_shape=[
            jax.ShapeDtypeStruct((E, D), jnp.float32),
            jax.ShapeDtypeStruct((E, HP), jnp.float32),
        ],
    )(k, v, qe, wk, bk, wv, bv)


SMW = LP * H


def _scatter_agg_body(vvu_hbm, idx_hbm, zrow_hbm, agg_hbm,
                      idx_a, idx_b, rows_a, rows_b, idx_t, rows_t, agg_sh,
                      s0, s1, s2, s3, s4, s5):
    cid = lax.axis_index("c")
    sid = lax.axis_index("s")
    wid = sid * 2 + cid
    base = wid * EDGES_PER_TILE
    row0 = sid * ROWS_PER_TILE

    for z in range(ROWS_PER_TILE // CHUNK):
        pltpu.sync_copy(zrow_hbm, agg_sh.at[pl.ds(row0 + z * CHUNK, CHUNK)])
    plsc.subcore_barrier()

    def pair(j, carry):
        off_a = base + (2 * j) * CHUNK
        off_b = off_a + CHUNK
        ia = pltpu.async_copy(idx_hbm.at[pl.ds(off_a, CHUNK)], idx_a, s0)
        ib = pltpu.async_copy(idx_hbm.at[pl.ds(off_b, CHUNK)], idx_b, s1)
        ra = pltpu.async_copy(vvu_hbm.at[pl.ds(off_a, CHUNK)], rows_a, s2)
        rb = pltpu.async_copy(vvu_hbm.at[pl.ds(off_b, CHUNK)], rows_b, s3)
        ia.wait()
        ra.wait()
        sa = pltpu.async_copy(rows_a, agg_sh.at[idx_a], s4, add=True)
        ib.wait()
        rb.wait()
        sb = pltpu.async_copy(rows_b, agg_sh.at[idx_b], s5, add=True)
        sa.wait()
        sb.wait()
        return carry

    lax.fori_loop(0, NFULL // 2, pair, 0)

    off = base + NFULL * CHUNK
    pltpu.sync_copy(idx_hbm.at[pl.ds(off, TAIL)], idx_t)
    pltpu.sync_copy(vvu_hbm.at[pl.ds(off, TAIL)], rows_t)
    pltpu.sync_copy(rows_t, agg_sh.at[idx_t], add=True)

    plsc.subcore_barrier()
    pltpu.sync_copy(agg_sh.at[pl.ds(row0, ROWS_PER_TILE)],
                    agg_hbm.at[cid, pl.ds(row0, ROWS_PER_TILE)])


def _scatter_agg(vvu, idx):
    mesh = plsc.VectorSubcoreMesh(core_axis_name="c", subcore_axis_name="s")
    zrow = jnp.zeros((CHUNK, D), jnp.float32)
    kern = functools.partial(
        pl.kernel,
        mesh=mesh,
        out_type=jax.ShapeDtypeStruct((2, LP, D), jnp.float32),
        scratch_types=[
            pltpu.VMEM((CHUNK,), jnp.int32),
            pltpu.VMEM((CHUNK,), jnp.int32),
            pltpu.VMEM((CHUNK, D), jnp.float32),
            pltpu.VMEM((CHUNK, D), jnp.float32),
            pltpu.VMEM((TAIL,), jnp.int32),
            pltpu.VMEM((TAIL, D), jnp.float32),
            pltpu.VMEM_SHARED((LP, D), jnp.float32),
            pltpu.SemaphoreType.DMA,
            pltpu.SemaphoreType.DMA,
            pltpu.SemaphoreType.DMA,
            pltpu.SemaphoreType.DMA,
            pltpu.SemaphoreType.DMA,
            pltpu.SemaphoreType.DMA,
        ],
    )(_scatter_agg_body)
    return kern(vvu, idx, zrow)


CHUNK_SM = 768
NFULL_SM = EDGES_PER_TILE // CHUNK_SM
TAIL_SM = EDGES_PER_TILE - NFULL_SM * CHUNK_SM


def _scatter_sm_body(exw_hbm, idx_hbm, sm_hbm, idx_v, ex_v, sm_t, sem):
    cid = lax.axis_index("c")
    sid = lax.axis_index("s")
    wid = sid * 2 + cid
    base = wid * EDGES_PER_TILE

    def zero(i, carry):
        sm_t[pl.ds(i * 16, 16)] = jnp.zeros((16,), jnp.float32)
        return carry

    lax.fori_loop(0, SMW // 16, zero, 0)

    def scatter_sm(nedges):
        for g in range(nedges // 16):
            rows = lax.iota(jnp.int32, 16) + g * 16
            nidx = idx_v[pl.ds(g * 16, 16)]
            for h in range(H):
                vals = plsc.load_gather(ex_v, [rows * HP + h])
                plsc.addupdate_scatter(sm_t, [nidx * H + h], vals)

    def body(c, carry):
        off = base + c * CHUNK_SM
        pltpu.sync_copy(idx_hbm.at[pl.ds(off, CHUNK_SM)], idx_v)
        pltpu.sync_copy(exw_hbm.at[pl.ds(off * HP, CHUNK_SM * HP)], ex_v)
        scatter_sm(CHUNK_SM)
        return carry

    lax.fori_loop(0, NFULL_SM, body, 0)

    off = base + NFULL_SM * CHUNK_SM
    pltpu.sync_copy(idx_hbm.at[pl.ds(off, TAIL_SM)], idx_v.at[pl.ds(0, TAIL_SM)])
    pltpu.sync_copy(exw_hbm.at[pl.ds(off * HP, TAIL_SM * HP)],
                    ex_v.at[pl.ds(0, TAIL_SM * HP)])
    scatter_sm(TAIL_SM)

    pltpu.sync_copy(sm_t, sm_hbm.at[wid])


def _scatter_sm(exw, idx):
    mesh = plsc.VectorSubcoreMesh(core_axis_name="c", subcore_axis_name="s")
    kern = functools.partial(
        pl.kernel,
        mesh=mesh,
        compiler_params=pltpu.CompilerParams(needs_layout_passes=False),
        out_type=jax.ShapeDtypeStruct((NTILES, SMW), jnp.float32),
        scratch_types=[
            pltpu.VMEM((CHUNK_SM,), jnp.int32),
            pltpu.VMEM((CHUNK_SM * HP,), jnp.float32),
            pltpu.VMEM((SMW,), jnp.float32),
            pltpu.SemaphoreType.DMA,
        ],
    )(_scatter_sm_body)
    return kern(exw.reshape(-1), idx)


def _rsm_body(a_ref, o_ref):
    s = jnp.sum(a_ref[...], axis=0)
    o_ref[...] = (1.0 / (s + 1e-7)).reshape(1, -1)


def _reduce_sm(sm32, block=8192):
    grid = SMW // block
    return pl.pallas_call(
        _rsm_body,
        grid=(grid,),
        in_specs=[pl.BlockSpec((NTILES, block), lambda i: (0, i))],
        out_specs=pl.BlockSpec((1, block), lambda i: (0, i)),
        out_shape=jax.ShapeDtypeStruct((1, SMW), jnp.float32),
    )(sm32)


def _final_body(agg_ref, r_ref, wo_ref, bo_ref, g_ref, b_ref, o_ref):
    agg = agg_ref[0] + agg_ref[1]
    b_ids = lax.broadcasted_iota(jnp.int32, (H, D), 0)
    d_ids = lax.broadcasted_iota(jnp.int32, (H, D), 1) // DH
    expand = jnp.where(b_ids == d_ids, 1.0, 0.0).astype(jnp.float32)
    mult = jnp.dot(r_ref[...], expand, precision=_PREC)
    out = jnp.dot(agg * mult, wo_ref[...], precision=_PREC) + bo_ref[...]
    mean = jnp.mean(out, axis=-1, keepdims=True)
    var = jnp.mean((out - mean) ** 2, axis=-1, keepdims=True)
    o_ref[...] = (out - mean) / jnp.sqrt(var + 1e-5) * g_ref[...] + b_ref[...]


def _final_stage(agg2, recip, wo, bo, gamma, beta, block=1000):
    grid = L // block
    return pl.pallas_call(
        _final_body,
        grid=(grid,),
        in_specs=[
            pl.BlockSpec((2, block, D), lambda i: (0, i, 0)),
            pl.BlockSpec((block, H), lambda i: (i, 0)),
            pl.BlockSpec((D, D), lambda i: (0, 0)),
            pl.BlockSpec((1, D), lambda i: (0, 0)),
            pl.BlockSpec((1, D), lambda i: (0, 0)),
            pl.BlockSpec((1, D), lambda i: (0, 0)),
        ],
        out_specs=pl.BlockSpec((block, D), lambda i: (i, 0)),
        out_shape=jax.ShapeDtypeStruct((L, D), jnp.float32),
    )(agg2, recip, wo, bo, gamma, beta)


def kernel(f, k, v, agg_idx, Wq, bq, Wk, bk, Wv, bv, Wo, bo, gamma, beta):
    idx = agg_idx.astype(jnp.int32)
    bq2 = bq.reshape(1, D)
    bk2 = bk.reshape(1, D)
    bv2 = bv.reshape(1, D)
    bo2 = bo.reshape(1, D)
    g2 = gamma.reshape(1, D)
    be2 = beta.reshape(1, D)

    q = _project(f, Wq, bq2, block=1000)
    qe = _gather_rows(q, idx)
    vvu, exw = _edge_stage(k, v, qe, Wk, bk2, Wv, bv2)
    agg2 = _scatter_agg(vvu, idx)
    sm32 = _scatter_sm(exw, idx)
    recip = _reduce_sm(sm32)
    agg2 = agg2[:, :L]
    recip = recip.reshape(LP, H)[:L]
    return _final_stage(agg2, recip, Wo, bo2, g2, be2)
